# baseline (speedup 1.0000x reference)
import jax
import jax.numpy as jnp
from jax import lax
from jax.experimental import pallas as pl
from jax.experimental.pallas import tpu as pltpu

_TAU = 0.07
_Q, _K, _D = 2048, 16384, 256
_BQ = 256
_BK = 2048
_NQ = _Q // _BQ
_NK = _K // _BK


def _bias_body(x_ref, b_ref):
    xx = x_ref[...]
    sqx = lax.dot_general(
        jnp.ones((1, _D), jnp.float32), xx * xx, (((1,), (1,)), ((), ())),
        preferred_element_type=jnp.float32,
        precision=lax.Precision.HIGHEST,
    )
    b_ref[...] = sqx * (-1.0 / _TAU)


def _fused_body(y_ref, x_ref, b_ref, o_ref, m_ref, s_ref, c_ref):
    j = pl.program_id(1)
    k = pl.program_id(2)
    dot = lax.dot_general(
        y_ref[...], x_ref[...], (((1,), (1,)), ((), ())),
        preferred_element_type=jnp.float32,
    )
    logits = dot * (2.0 / _TAU) + b_ref[...]

    @pl.when(j == 0)
    def _stats():
        m_c = jnp.max(logits, axis=1, keepdims=True)
        e = jnp.exp(logits - m_c)
        s_c = jnp.sum(e, axis=1, keepdims=True)
        m_ref[k] = jnp.broadcast_to(m_c, (_BQ, 128))
        s_ref[k] = jnp.broadcast_to(s_c, (_BQ, 128))

        @pl.when(k == _NK - 1)
        def _merge():
            m_all = m_ref[...]
            s_all = s_ref[...]
            m = jnp.max(m_all, axis=0)
            s = jnp.sum(s_all * jnp.exp(m_all - m), axis=0)
            c_ref[...] = m + jnp.log(s)

    @pl.when(j == 1)
    def _emit():
        o_ref[...] = jnp.exp(logits - c_ref[:, 0:1])


def kernel(X, Y):
    bias = pl.pallas_call(
        _bias_body,
        grid=(1,),
        in_specs=[pl.BlockSpec((_K, _D), lambda i: (0, 0))],
        out_specs=pl.BlockSpec((1, _K), lambda i: (0, 0)),
        out_shape=jax.ShapeDtypeStruct((1, _K), jnp.float32),
    )(X)
    Xb = X.astype(jnp.bfloat16)
    Yb = Y.astype(jnp.bfloat16)
    return pl.pallas_call(
        _fused_body,
        grid=(_NQ, 2, _NK),
        in_specs=[
            pl.BlockSpec((_BQ, _D), lambda q, j, k: (q, 0)),
            pl.BlockSpec((_BK, _D), lambda q, j, k: (k, 0)),
            pl.BlockSpec((1, _BK), lambda q, j, k: (0, k)),
        ],
        out_specs=pl.BlockSpec((_BQ, _BK), lambda q, j, k: (q, j * k)),
        out_shape=jax.ShapeDtypeStruct((_Q, _K), jnp.float32),
        scratch_shapes=[
            pltpu.VMEM((_NK, _BQ, 128), jnp.float32),
            pltpu.VMEM((_NK, _BQ, 128), jnp.float32),
            pltpu.VMEM((_BQ, 128), jnp.float32),
        ],
        compiler_params=pltpu.CompilerParams(
            dimension_semantics=("parallel", "arbitrary", "arbitrary"),
        ),
    )(Yb, Xb, bias)


# speedup vs baseline: 1.5415x; 1.5415x over previous
import jax
import jax.numpy as jnp
from jax import lax
from jax.experimental import pallas as pl
from jax.experimental.pallas import tpu as pltpu

_TAU = 0.07
_Q, _K, _D = 2048, 16384, 256
_BQ = 256
_BK = 4096
_NQ = _Q // _BQ
_NK = _K // _BK


def _bias_body(x_ref, b_ref):
    xx = x_ref[...]
    sqx = lax.dot_general(
        jnp.ones((1, _D), jnp.float32), xx * xx, (((1,), (1,)), ((), ())),
        preferred_element_type=jnp.float32,
        precision=lax.Precision.HIGHEST,
    )
    b_ref[...] = sqx * (-1.0 / _TAU)


def _fused_body(y_ref, x_ref, b_ref, o_ref, m_ref, s_ref):
    k = pl.program_id(1)
    sl = pl.ds(k * _BK, _BK)
    dot = lax.dot_general(
        y_ref[...], x_ref[sl, :], (((1,), (1,)), ((), ())),
        preferred_element_type=jnp.float32,
    )
    logits = dot * (2.0 / _TAU) + b_ref[:, sl]

    m_c = jnp.max(logits, axis=1, keepdims=True)
    e = jnp.exp(logits - m_c)
    s_c = jnp.sum(e, axis=1, keepdims=True)

    o_ref[:, sl] = e
    m_ref[k] = jnp.broadcast_to(m_c, (_BQ, 128))
    s_ref[k] = jnp.broadcast_to(s_c, (_BQ, 128))

    @pl.when(k == _NK - 1)
    def _finalize():
        m_all = m_ref[...]
        s_all = s_ref[...]
        m = jnp.max(m_all, axis=0)
        w = jnp.exp(m_all - m)
        s = jnp.sum(s_all * w, axis=0)
        r = w / s
        for c in range(_NK):
            csl = pl.ds(c * _BK, _BK)
            o_ref[:, csl] = o_ref[:, csl] * r[c, :, 0:1]


def kernel(X, Y):
    bias = pl.pallas_call(
        _bias_body,
        grid=(1,),
        in_specs=[pl.BlockSpec((_K, _D), lambda i: (0, 0))],
        out_specs=pl.BlockSpec((1, _K), lambda i: (0, 0)),
        out_shape=jax.ShapeDtypeStruct((1, _K), jnp.float32),
    )(X)
    Xb = X.astype(jnp.bfloat16)
    Yb = Y.astype(jnp.bfloat16)
    return pl.pallas_call(
        _fused_body,
        grid=(_NQ, _NK),
        in_specs=[
            pl.BlockSpec((_BQ, _D), lambda q, k: (q, 0)),
            pl.BlockSpec((_K, _D), lambda q, k: (0, 0)),
            pl.BlockSpec((1, _K), lambda q, k: (0, 0)),
        ],
        out_specs=pl.BlockSpec((_BQ, _K), lambda q, k: (q, 0)),
        out_shape=jax.ShapeDtypeStruct((_Q, _K), jnp.float32),
        scratch_shapes=[
            pltpu.VMEM((_NK, _BQ, 128), jnp.float32),
            pltpu.VMEM((_NK, _BQ, 128), jnp.float32),
        ],
        compiler_params=pltpu.CompilerParams(
            dimension_semantics=("parallel", "arbitrary"),
        ),
    )(Yb, Xb, bias)


# speedup vs baseline: 1.6517x; 1.0715x over previous
import jax
import jax.numpy as jnp
from jax import lax
from jax.experimental import pallas as pl
from jax.experimental.pallas import tpu as pltpu

_TAU = 0.07
_Q, _K, _D = 2048, 16384, 256
_BQ = 256
_BK = 4096
_NQ = _Q // _BQ
_NK = _K // _BK


def _bias_body(x_ref, b_ref):
    xx = x_ref[...]
    p = xx * xx
    p_hi = p.astype(jnp.bfloat16)
    p_lo = (p - p_hi.astype(jnp.float32)).astype(jnp.bfloat16)
    ones = jnp.ones((1, _D), jnp.bfloat16)
    dn = (((1,), (1,)), ((), ()))
    sqx = (
        lax.dot_general(ones, p_hi, dn, preferred_element_type=jnp.float32)
        + lax.dot_general(ones, p_lo, dn, preferred_element_type=jnp.float32)
    )
    b_ref[...] = sqx * (-1.0 / _TAU)


def _fused_body(y_ref, x_ref, b_ref, o_ref, m_ref, s_ref):
    k = pl.program_id(1)
    sl = pl.ds(k * _BK, _BK)
    dot = lax.dot_general(
        y_ref[...], x_ref[sl, :], (((1,), (1,)), ((), ())),
        preferred_element_type=jnp.float32,
    )
    logits = dot * (2.0 / _TAU) + b_ref[:, sl]

    m_c = jnp.max(logits, axis=1, keepdims=True)
    e = jnp.exp(logits - m_c)
    s_c = jnp.sum(e, axis=1, keepdims=True)

    o_ref[:, sl] = e
    m_ref[k] = jnp.broadcast_to(m_c, (_BQ, 128))
    s_ref[k] = jnp.broadcast_to(s_c, (_BQ, 128))

    @pl.when(k == _NK - 1)
    def _finalize():
        m_all = m_ref[...]
        s_all = s_ref[...]
        m = jnp.max(m_all, axis=0)
        w = jnp.exp(m_all - m)
        s = jnp.sum(s_all * w, axis=0)
        r = w / s
        for c in range(_NK):
            csl = pl.ds(c * _BK, _BK)
            o_ref[:, csl] = o_ref[:, csl] * r[c, :, 0:1]


def kernel(X, Y):
    bias = pl.pallas_call(
        _bias_body,
        grid=(1,),
        in_specs=[pl.BlockSpec((_K, _D), lambda i: (0, 0))],
        out_specs=pl.BlockSpec((1, _K), lambda i: (0, 0)),
        out_shape=jax.ShapeDtypeStruct((1, _K), jnp.float32),
    )(X)
    Xb = X.astype(jnp.bfloat16)
    Yb = Y.astype(jnp.bfloat16)
    return pl.pallas_call(
        _fused_body,
        grid=(_NQ, _NK),
        in_specs=[
            pl.BlockSpec((_BQ, _D), lambda q, k: (q, 0)),
            pl.BlockSpec((_K, _D), lambda q, k: (0, 0)),
            pl.BlockSpec((1, _K), lambda q, k: (0, 0)),
        ],
        out_specs=pl.BlockSpec((_BQ, _K), lambda q, k: (q, 0)),
        out_shape=jax.ShapeDtypeStruct((_Q, _K), jnp.float32),
        scratch_shapes=[
            pltpu.VMEM((_NK, _BQ, 128), jnp.float32),
            pltpu.VMEM((_NK, _BQ, 128), jnp.float32),
        ],
        compiler_params=pltpu.CompilerParams(
            dimension_semantics=("parallel", "arbitrary"),
        ),
    )(Yb, Xb, bias)


# speedup vs baseline: 1.8741x; 1.1346x over previous
import jax
import jax.numpy as jnp
from jax import lax
from jax.experimental import pallas as pl
from jax.experimental.pallas import tpu as pltpu

_TAU = 0.07
_Q, _K, _D = 2048, 16384, 256
_BQ = 256
_BK = 8192
_NQ = _Q // _BQ
_NK = _K // _BK


def _bias_body(x_ref, b_ref):
    xx = x_ref[...]
    p = xx * xx
    p_hi = p.astype(jnp.bfloat16)
    p_lo = (p - p_hi.astype(jnp.float32)).astype(jnp.bfloat16)
    ones = jnp.ones((1, _D), jnp.bfloat16)
    dn = (((1,), (1,)), ((), ()))
    sqx = (
        lax.dot_general(ones, p_hi, dn, preferred_element_type=jnp.float32)
        + lax.dot_general(ones, p_lo, dn, preferred_element_type=jnp.float32)
    )
    b_ref[...] = sqx * (-1.0 / _TAU)


def _fused_body(y_ref, x_ref, b_ref, o_ref, m_ref, s_ref):
    k = pl.program_id(1)
    sl = pl.ds(k * _BK, _BK)
    dot = lax.dot_general(
        y_ref[...], x_ref[sl, :], (((1,), (1,)), ((), ())),
        preferred_element_type=jnp.float32,
    )
    logits = dot * (2.0 / _TAU) + b_ref[:, sl]

    m_c = jnp.max(logits, axis=1, keepdims=True)
    e = jnp.exp(logits - m_c)
    s_c = jnp.sum(e, axis=1, keepdims=True)

    o_ref[:, sl] = e
    m_ref[k] = jnp.broadcast_to(m_c, (_BQ, 128))
    s_ref[k] = jnp.broadcast_to(s_c, (_BQ, 128))

    @pl.when(k == _NK - 1)
    def _finalize():
        m_all = m_ref[...]
        s_all = s_ref[...]
        m = jnp.max(m_all, axis=0)
        w = jnp.exp(m_all - m)
        s = jnp.sum(s_all * w, axis=0)
        r = w / s
        for c in range(_NK):
            csl = pl.ds(c * _BK, _BK)
            o_ref[:, csl] = o_ref[:, csl] * r[c, :, 0:1]


def kernel(X, Y):
    bias = pl.pallas_call(
        _bias_body,
        grid=(1,),
        in_specs=[pl.BlockSpec((_K, _D), lambda i: (0, 0))],
        out_specs=pl.BlockSpec((1, _K), lambda i: (0, 0)),
        out_shape=jax.ShapeDtypeStruct((1, _K), jnp.float32),
    )(X)
    Xb = X.astype(jnp.bfloat16)
    Yb = Y.astype(jnp.bfloat16)
    return pl.pallas_call(
        _fused_body,
        grid=(_NQ, _NK),
        in_specs=[
            pl.BlockSpec((_BQ, _D), lambda q, k: (q, 0)),
            pl.BlockSpec((_K, _D), lambda q, k: (0, 0)),
            pl.BlockSpec((1, _K), lambda q, k: (0, 0)),
        ],
        out_specs=pl.BlockSpec((_BQ, _K), lambda q, k: (q, 0)),
        out_shape=jax.ShapeDtypeStruct((_Q, _K), jnp.float32),
        scratch_shapes=[
            pltpu.VMEM((_NK, _BQ, 128), jnp.float32),
            pltpu.VMEM((_NK, _BQ, 128), jnp.float32),
        ],
        compiler_params=pltpu.CompilerParams(
            dimension_semantics=("parallel", "arbitrary"),
        ),
    )(Yb, Xb, bias)


# speedup vs baseline: 2.3696x; 1.2644x over previous
import jax
import jax.numpy as jnp
from jax import lax
from jax.experimental import pallas as pl
from jax.experimental.pallas import tpu as pltpu

_TAU = 0.07
_Q, _K, _D = 2048, 16384, 256
_BQ = 256
_BK = 16384
_NQ = _Q // _BQ
_NK = _K // _BK


def _bias_body(x_ref, b_ref):
    xx = x_ref[...]
    p = xx * xx
    p_hi = p.astype(jnp.bfloat16)
    p_lo = (p - p_hi.astype(jnp.float32)).astype(jnp.bfloat16)
    ones = jnp.ones((1, _D), jnp.bfloat16)
    dn = (((1,), (1,)), ((), ()))
    sqx = (
        lax.dot_general(ones, p_hi, dn, preferred_element_type=jnp.float32)
        + lax.dot_general(ones, p_lo, dn, preferred_element_type=jnp.float32)
    )
    b_ref[...] = sqx * (-1.0 / _TAU)


def _fused_body(y_ref, x_ref, b_ref, o_ref, m_ref, s_ref):
    k = pl.program_id(1)
    sl = pl.ds(k * _BK, _BK)
    dot = lax.dot_general(
        y_ref[...], x_ref[sl, :], (((1,), (1,)), ((), ())),
        preferred_element_type=jnp.float32,
    )
    logits = dot * (2.0 / _TAU) + b_ref[:, sl]

    m_c = jnp.max(logits, axis=1, keepdims=True)
    e = jnp.exp(logits - m_c)
    s_c = jnp.sum(e, axis=1, keepdims=True)

    o_ref[:, sl] = e
    m_ref[k] = jnp.broadcast_to(m_c, (_BQ, 128))
    s_ref[k] = jnp.broadcast_to(s_c, (_BQ, 128))

    @pl.when(k == _NK - 1)
    def _finalize():
        m_all = m_ref[...]
        s_all = s_ref[...]
        m = jnp.max(m_all, axis=0)
        w = jnp.exp(m_all - m)
        s = jnp.sum(s_all * w, axis=0)
        r = w / s
        for c in range(_NK):
            csl = pl.ds(c * _BK, _BK)
            o_ref[:, csl] = o_ref[:, csl] * r[c, :, 0:1]


def kernel(X, Y):
    bias = pl.pallas_call(
        _bias_body,
        grid=(1,),
        in_specs=[pl.BlockSpec((_K, _D), lambda i: (0, 0))],
        out_specs=pl.BlockSpec((1, _K), lambda i: (0, 0)),
        out_shape=jax.ShapeDtypeStruct((1, _K), jnp.float32),
    )(X)
    Xb = X.astype(jnp.bfloat16)
    Yb = Y.astype(jnp.bfloat16)
    return pl.pallas_call(
        _fused_body,
        grid=(_NQ, _NK),
        in_specs=[
            pl.BlockSpec((_BQ, _D), lambda q, k: (q, 0)),
            pl.BlockSpec((_K, _D), lambda q, k: (0, 0)),
            pl.BlockSpec((1, _K), lambda q, k: (0, 0)),
        ],
        out_specs=pl.BlockSpec((_BQ, _K), lambda q, k: (q, 0)),
        out_shape=jax.ShapeDtypeStruct((_Q, _K), jnp.float32),
        scratch_shapes=[
            pltpu.VMEM((_NK, _BQ, 128), jnp.float32),
            pltpu.VMEM((_NK, _BQ, 128), jnp.float32),
        ],
        compiler_params=pltpu.CompilerParams(
            dimension_semantics=("parallel", "arbitrary"),
        ),
    )(Yb, Xb, bias)


# speedup vs baseline: 2.3904x; 1.0088x over previous
import jax
import jax.numpy as jnp
from jax import lax
from jax.experimental import pallas as pl
from jax.experimental.pallas import tpu as pltpu

_TAU = 0.07
_Q, _K, _D = 2048, 16384, 256
_BQ = 256
_NQ = _Q // _BQ


def _bias_body(x_ref, b_ref):
    xx = x_ref[...]
    p = xx * xx
    p_hi = p.astype(jnp.bfloat16)
    p_lo = (p - p_hi.astype(jnp.float32)).astype(jnp.bfloat16)
    ones = jnp.ones((1, _D), jnp.bfloat16)
    dn = (((1,), (1,)), ((), ()))
    sqx = (
        lax.dot_general(ones, p_hi, dn, preferred_element_type=jnp.float32)
        + lax.dot_general(ones, p_lo, dn, preferred_element_type=jnp.float32)
    )
    b_ref[...] = sqx * (-1.0 / _TAU)


def _fused_body(y_ref, x_ref, b_ref, o_ref):
    dot = lax.dot_general(
        y_ref[...], x_ref[...], (((1,), (1,)), ((), ())),
        preferred_element_type=jnp.float32,
    )
    logits = dot * (2.0 / _TAU) + b_ref[...]

    m = jnp.max(logits, axis=1, keepdims=True)
    e = jnp.exp(logits - m)
    s = jnp.sum(e, axis=1, keepdims=True)
    o_ref[...] = e
    o_ref[...] = o_ref[...] * (1.0 / s)


def kernel(X, Y):
    bias = pl.pallas_call(
        _bias_body,
        grid=(1,),
        in_specs=[pl.BlockSpec((_K, _D), lambda i: (0, 0))],
        out_specs=pl.BlockSpec((1, _K), lambda i: (0, 0)),
        out_shape=jax.ShapeDtypeStruct((1, _K), jnp.float32),
    )(X)
    Xb = X.astype(jnp.bfloat16)
    Yb = Y.astype(jnp.bfloat16)
    return pl.pallas_call(
        _fused_body,
        grid=(_NQ,),
        in_specs=[
            pl.BlockSpec((_BQ, _D), lambda q: (q, 0)),
            pl.BlockSpec((_K, _D), lambda q: (0, 0)),
            pl.BlockSpec((1, _K), lambda q: (0, 0)),
        ],
        out_specs=pl.BlockSpec((_BQ, _K), lambda q: (q, 0)),
        out_shape=jax.ShapeDtypeStruct((_Q, _K), jnp.float32),
        compiler_params=pltpu.CompilerParams(
            dimension_semantics=("arbitrary",),
        ),
    )(Yb, Xb, bias)
